# Initial kernel scaffold; baseline (speedup 1.0000x reference)
import jax
import jax.numpy as jnp
from jax.experimental import pallas as pl


def kernel(pos, rgb, batch, W1_1, b1_1, W2_1, b2_1, W1_2, b1_2, W2_2, b2_2, W1_3, b1_3, W2_3, b2_3):
    raise NotImplementedError("write your pallas kernel here")


# speedup vs baseline: 38.2777x; 38.2777x over previous
import functools

import jax
import jax.numpy as jnp
from jax import lax
from jax.experimental import pallas as pl
from jax.experimental.pallas import tpu as pltpu
from jax.experimental.pallas import tpu_sc as plsc

N = 16384
B = 8
NB = N // B
KNB = 64
R2 = (1.0 / 16.0) ** 2
NWORK = 32
RW = N // NWORK
SENT = N
APAD = N + 128
INF_BITS = 0x7F800000


def _d2_body(pos_ref, o_ref):
    p = pos_ref[...]
    sq = jnp.sum(p * p, axis=1)
    dot = lax.dot_general(p, p, (((1,), (1,)), ((), ())),
                          preferred_element_type=jnp.float32)
    o_ref[...] = jnp.maximum(sq[:, None] + sq[None, :] - 2.0 * dot, 0.0)


def _d2_call(pos):
    return pl.pallas_call(
        _d2_body,
        grid=(B,),
        in_specs=[pl.BlockSpec((NB, 3), lambda b: (b, 0))],
        out_specs=pl.BlockSpec((NB, NB), lambda b: (b, 0)),
        out_shape=jax.ShapeDtypeStruct((N, NB), jnp.float32),
    )(pos)


def _nbr_body(d2_hbm, nbr_hbm, dbuf, cand_idx, cand_d2, hist, outflat,
              sem0, sem1, osem):
    wid = lax.axis_index("s") * 2 + lax.axis_index("c")
    base_row = wid * RW
    batch_base = (base_row // NB) * NB

    lane = lax.iota(jnp.int32, 16)
    ones16 = jnp.ones((16,), jnp.int32)

    def row_dma(r, slot, sem):
        return pltpu.make_async_copy(
            d2_hbm.at[pl.ds(base_row + r, 1)], dbuf.at[slot], sem)

    def process(r, slot):
        dref = dbuf.at[slot]

        def comp_body(v, cnt):
            d = dref[0, pl.ds(v * 16, 16)]
            msk = d <= R2
            mi = jnp.where(msk, 1, 0)
            pos_v = plsc.cumsum(mi) + (cnt - 1)
            jvec = lane + v * 16
            plsc.store_scatter(cand_idx, [pos_v], jvec, mask=msk)
            plsc.store_scatter(cand_d2, [pos_v], d, mask=msk)
            return cnt + jnp.sum(mi)

        m = lax.fori_loop(0, NB // 16, comp_body, jnp.int32(0))

        inf16 = jnp.full((16,), jnp.inf, jnp.float32)
        plsc.store_scatter(cand_d2, [m + lane], inf16)
        nv = (m + 15) // 16

        def do_select():
            pref = jnp.int32(0)
            target = jnp.int32(KNB)
            for pi, shift in enumerate((24, 16, 8, 0)):
                def clr(h, _):
                    hist[pl.ds(h * 16, 16)] = jnp.zeros((16,), jnp.int32)
                    return 0
                lax.fori_loop(0, 16, clr, 0)

                def hbody(v, _, shift=shift, pi=pi, pref=pref):
                    d = cand_d2[pl.ds(v * 16, 16)]
                    bits = plsc.bitcast(d, jnp.int32)
                    binv = jnp.right_shift(bits, shift) & 0xFF
                    if pi == 0:
                        ok = bits < jnp.int32(INF_BITS)
                    else:
                        hi = jnp.right_shift(bits, shift + 8)
                        ok = hi == pref
                    plsc.addupdate_scatter(hist, [binv], ones16, mask=ok)
                    return 0
                lax.fori_loop(0, nv, hbody, 0)

                def sbody(h, carry, target=target):
                    run, bin_, cb, found = carry
                    hv = hist[pl.ds(h * 16, 16)]
                    s_h = jnp.sum(hv)
                    c = plsc.cumsum(hv) + run
                    ge = c >= target
                    np_ = jnp.sum(jnp.where(ge, 1, 0))
                    ffs = 16 - np_
                    found_here = jnp.logical_and(jnp.logical_not(found),
                                                 np_ > 0)
                    cb_h = run + jnp.sum(jnp.where(lane < ffs, hv, 0))
                    bin_h = h * 16 + ffs
                    bin_ = jnp.where(found_here, bin_h, bin_)
                    cb = jnp.where(found_here, cb_h, cb)
                    return (run + s_h, bin_, cb,
                            jnp.logical_or(found, np_ > 0))

                _, bin_k, cb_k, _ = lax.fori_loop(
                    0, 16, sbody,
                    (jnp.int32(0), jnp.int32(0), jnp.int32(0),
                     jnp.bool_(False)))
                if pi == 0:
                    pref = bin_k
                else:
                    pref = (pref << 8) | bin_k
                target = target - cb_k
            return pref, target

        t_bits, q = lax.cond(
            m > KNB, do_select,
            lambda: (jnp.int32(INF_BITS), jnp.int32(0)))

        obase = r * KNB
        for v in range(KNB // 16):
            outflat[pl.ds(obase + v * 16, 16)] = jnp.full((16,), SENT,
                                                          jnp.int32)

        def ebody(v, carry):
            re, kp = carry
            d = cand_d2[pl.ds(v * 16, 16)]
            bits = plsc.bitcast(d, jnp.int32)
            lt = bits < t_bits
            eq = bits == t_bits
            eqr = plsc.cumsum(jnp.where(eq, 1, 0)) + re
            keep = jnp.logical_or(lt, jnp.logical_and(eq, eqr <= q))
            ki = jnp.where(keep, 1, 0)
            pos_v = plsc.cumsum(ki) + (kp - 1) + obase
            gidx = cand_idx[pl.ds(v * 16, 16)] + batch_base
            plsc.store_scatter(outflat, [pos_v], gidx, mask=keep)
            return re + jnp.sum(jnp.where(eq, 1, 0)), kp + jnp.sum(ki)

        lax.fori_loop(0, nv, ebody, (jnp.int32(0), jnp.int32(0)))

    row_dma(0, 0, sem0).start()
    row_dma(1, 1, sem1).start()

    def pair_body(rr, _):
        r0 = rr * 2
        row_dma(r0, 0, sem0).wait()
        process(r0, 0)

        @pl.when(rr < RW // 2 - 1)
        def _():
            row_dma(r0 + 2, 0, sem0).start()

        row_dma(r0 + 1, 1, sem1).wait()
        process(r0 + 1, 1)

        @pl.when(rr < RW // 2 - 1)
        def _():
            row_dma(r0 + 3, 1, sem1).start()
        return 0

    lax.fori_loop(0, RW // 2, pair_body, 0)

    cp = pltpu.make_async_copy(
        outflat, nbr_hbm.at[pl.ds(base_row * KNB, RW * KNB)], osem)
    cp.start()
    cp.wait()


def _nbr_call(d2_all):
    mesh = plsc.VectorSubcoreMesh(core_axis_name="c", subcore_axis_name="s")
    f = pl.kernel(
        _nbr_body,
        mesh=mesh,
        out_type=jax.ShapeDtypeStruct((N * KNB,), jnp.int32),
        scratch_types=[
            pltpu.VMEM((2, 1, NB), jnp.float32),
            pltpu.VMEM((NB + 32,), jnp.int32),
            pltpu.VMEM((NB + 32,), jnp.float32),
            pltpu.VMEM((256,), jnp.int32),
            pltpu.VMEM((RW * KNB,), jnp.int32),
            pltpu.SemaphoreType.DMA,
            pltpu.SemaphoreType.DMA,
            pltpu.SemaphoreType.DMA,
        ],
        compiler_params=pltpu.CompilerParams(needs_layout_passes=False),
    )
    return f(d2_all)


def _gmax_body(a_hbm, c_hbm, nbr_hbm, agg_hbm, nbrbuf, gbuf, cbuf, obuf,
               nsem, csem, sem0, sem1, osem, *, dp):
    wid = lax.axis_index("s") * 2 + lax.axis_index("c")
    base_row = wid * RW

    cpn = pltpu.make_async_copy(
        nbr_hbm.at[pl.ds(base_row * KNB, RW * KNB)], nbrbuf, nsem)
    cpn.start()
    cpc = pltpu.make_async_copy(c_hbm.at[pl.ds(base_row, RW)], cbuf, csem)
    cpc.start()
    cpn.wait()
    cpc.wait()

    def gather_dma(r, slot, sem):
        return pltpu.make_async_copy(
            a_hbm.at[nbrbuf.at[pl.ds((r - base_row) * KNB, KNB)]],
            gbuf.at[slot], sem)

    def process(r, slot):
        rl = r - base_row
        nk = dp // 16
        cvecs = [cbuf[rl, pl.ds(k * 16, 16)] for k in range(nk)]

        def jbody(j, acc):
            return tuple(
                jnp.maximum(acc[k],
                            gbuf[slot, j, pl.ds(k * 16, 16)] + cvecs[k])
                for k in range(nk))

        acc0 = tuple(jnp.zeros((16,), jnp.float32) for _ in range(nk))
        acc = lax.fori_loop(0, KNB, jbody, acc0)
        for k in range(nk):
            obuf[rl, pl.ds(k * 16, 16)] = acc[k]

    gather_dma(base_row, 0, sem0).start()
    gather_dma(base_row + 1, 1, sem1).start()

    def pair_body(rr, _):
        r0 = base_row + rr * 2
        gather_dma(r0, 0, sem0).wait()
        process(r0, 0)

        @pl.when(rr < RW // 2 - 1)
        def _():
            gather_dma(r0 + 2, 0, sem0).start()

        gather_dma(r0 + 1, 1, sem1).wait()
        process(r0 + 1, 1)

        @pl.when(rr < RW // 2 - 1)
        def _():
            gather_dma(r0 + 3, 1, sem1).start()
        return 0

    lax.fori_loop(0, RW // 2, pair_body, 0)

    cp = pltpu.make_async_copy(obuf, agg_hbm.at[pl.ds(base_row, RW)], osem)
    cp.start()
    cp.wait()


def _gmax_call(a_table, c_table, nbr, dp):
    mesh = plsc.VectorSubcoreMesh(core_axis_name="c", subcore_axis_name="s")
    f = pl.kernel(
        functools.partial(_gmax_body, dp=dp),
        mesh=mesh,
        out_type=jax.ShapeDtypeStruct((N, dp), jnp.float32),
        scratch_types=[
            pltpu.VMEM((RW * KNB,), jnp.int32),
            pltpu.VMEM((2, KNB, dp), jnp.float32),
            pltpu.VMEM((RW, dp), jnp.float32),
            pltpu.VMEM((RW, dp), jnp.float32),
            pltpu.SemaphoreType.DMA,
            pltpu.SemaphoreType.DMA,
            pltpu.SemaphoreType.DMA,
            pltpu.SemaphoreType.DMA,
            pltpu.SemaphoreType.DMA,
        ],
        compiler_params=pltpu.CompilerParams(
            needs_layout_passes=False, use_tc_tiling_on_sc=False),
    )
    return f(a_table, c_table, nbr)


def _celu(x):
    return jnp.where(x > 0.0, x, jnp.exp(jnp.minimum(x, 0.0)) - 1.0)


def _pad_cols(x, dp):
    c = x.shape[1]
    if c == dp:
        return x
    return jnp.concatenate(
        [x, jnp.zeros((x.shape[0], dp - c), x.dtype)], axis=1)


def _pre_body(pos_ref, w1s_ref, b1_ref, a1_ref, c1_ref):
    p = pos_ref[...]
    w1s = w1s_ref[...]
    b1 = b1_ref[...]
    dp1 = a1_ref.shape[1]
    both = lax.dot_general(p, w1s, (((1,), (0,)), ((), ())),
                           preferred_element_type=jnp.float32)
    a1_ref[pl.ds(0, N), :] = both[:, :dp1] + b1
    a1_ref[pl.ds(N, APAD - N), :] = jnp.full((APAD - N, dp1), -1e30,
                                             jnp.float32)
    c1_ref[...] = both[:, dp1:]


def _pre_call(pos, w1s, b1, dp1):
    return pl.pallas_call(
        _pre_body,
        out_shape=(jax.ShapeDtypeStruct((APAD, dp1), jnp.float32),
                   jax.ShapeDtypeStruct((N, dp1), jnp.float32)),
    )(pos, w1s, b1)


def _step_body(agg_ref, w2_ref, b2_ref, w1n_ref, pn_ref, an_ref, *, cmid):
    agg = agg_ref[...][:, :cmid]
    o = _celu(lax.dot_general(agg, w2_ref[...], (((1,), (0,)), ((), ())),
                              preferred_element_type=jnp.float32)
              + b2_ref[...])
    a_n = lax.dot_general(o, w1n_ref[...], (((1,), (0,)), ((), ())),
                          preferred_element_type=jnp.float32) + pn_ref[...]
    dpn = an_ref.shape[1]
    an_ref[pl.ds(0, N), :] = a_n
    an_ref[pl.ds(N, APAD - N), :] = jnp.full((APAD - N, dpn), -1e30,
                                             jnp.float32)


def _step_call(agg, w2, b2, w1n, pn, cmid, dpn):
    return pl.pallas_call(
        functools.partial(_step_body, cmid=cmid),
        out_shape=jax.ShapeDtypeStruct((APAD, dpn), jnp.float32),
    )(agg, w2, b2, w1n, pn)


def _final_body(agg_ref, w2_ref, b2_ref, out_ref, *, cmid):
    agg = agg_ref[...][:, :cmid]
    out_ref[...] = _celu(
        lax.dot_general(agg, w2_ref[...], (((1,), (0,)), ((), ())),
                        preferred_element_type=jnp.float32) + b2_ref[...])


def _final_call(agg, w2, b2, cmid, cout):
    return pl.pallas_call(
        functools.partial(_final_body, cmid=cmid),
        out_shape=jax.ShapeDtypeStruct((N, cout), jnp.float32),
    )(agg, w2, b2)


def _pn_body(pos_ref, w1p_ref, b1_ref, pn_ref, cn_ref):
    p = pos_ref[...]
    w1p = w1p_ref[...]
    pn = lax.dot_general(p, w1p, (((1,), (0,)), ((), ())),
                         preferred_element_type=jnp.float32)
    pn_ref[...] = pn + b1_ref[...]
    cn_ref[...] = -pn


def _pn_call(pos, w1p, b1, dp):
    return pl.pallas_call(
        _pn_body,
        out_shape=(jax.ShapeDtypeStruct((N, dp), jnp.float32),
                   jax.ShapeDtypeStruct((N, dp), jnp.float32)),
    )(pos, w1p, b1)


def kernel(pos, rgb, batch,
           W1_1, b1_1, W2_1, b2_1,
           W1_2, b1_2, W2_2, b2_2,
           W1_3, b1_3, W2_3, b2_3):
    del rgb
    dp1, dp2, dp3 = 16, 16, 32
    c1, c2, c3 = 8, 16, 32

    d2_all = _d2_call(pos)
    nbr = _nbr_call(d2_all)

    w1a, w1b = W1_1[:3], W1_1[3:]
    w1s = jnp.concatenate(
        [_pad_cols(w1a + w1b, dp1), _pad_cols(-w1b, dp1)], axis=1)
    a1, cc1 = _pre_call(pos, w1s, _pad_cols(b1_1[None, :], dp1), dp1)

    p2, cc2 = _pn_call(pos, _pad_cols(W1_2[c1:], dp2),
                       _pad_cols(b1_2[None, :], dp2), dp2)
    p3, cc3 = _pn_call(pos, _pad_cols(W1_3[c2:], dp3),
                       _pad_cols(b1_3[None, :], dp3), dp3)

    agg1 = _gmax_call(a1, cc1, nbr, dp1)
    a2 = _step_call(agg1, W2_1, b2_1[None, :],
                    _pad_cols(W1_2[:c1], dp2), p2, c1, dp2)
    agg2 = _gmax_call(a2, cc2, nbr, dp2)
    a3 = _step_call(agg2, W2_2, b2_2[None, :],
                    _pad_cols(W1_3[:c2], dp3), p3, c2, dp3)
    agg3 = _gmax_call(a3, cc3, nbr, dp3)
    out = _final_call(agg3, W2_3, b2_3[None, :], c3, c3)

    return (pos, out, batch)


# speedup vs baseline: 41.5863x; 1.0864x over previous
import functools

import jax
import jax.numpy as jnp
from jax import lax
from jax.experimental import pallas as pl
from jax.experimental.pallas import tpu as pltpu
from jax.experimental.pallas import tpu_sc as plsc

N = 16384
B = 8
NB = N // B
KNB = 64
R2 = (1.0 / 16.0) ** 2
NWORK = 32
RW = N // NWORK
SENT = N
APAD = N + 128
INF_BITS = 0x7F800000


def _d2_body(pos_ref, o_ref):
    p = pos_ref[...]
    x, y, z = p[:, 0], p[:, 1], p[:, 2]
    sq = (x * x + z * z) + y * y
    pb = p.astype(jnp.bfloat16)
    dot = lax.dot_general(pb, pb, (((1,), (1,)), ((), ())),
                          preferred_element_type=jnp.float32)
    o_ref[...] = jnp.maximum(sq[:, None] + sq[None, :] - 2.0 * dot, 0.0)


def _d2_call(pos):
    return pl.pallas_call(
        _d2_body,
        grid=(B,),
        in_specs=[pl.BlockSpec((NB, 3), lambda b: (b, 0))],
        out_specs=pl.BlockSpec((NB, NB), lambda b: (b, 0)),
        out_shape=jax.ShapeDtypeStruct((N, NB), jnp.float32),
    )(pos)


def _nbr_body(d2_hbm, nbr_hbm, dbuf, cand_idx, cand_d2, eqbuf, hist, ebuf,
              outflat, sem0, sem1, osem):
    wid = lax.axis_index("s") * 2 + lax.axis_index("c")
    base_row = wid * RW
    batch_base = (base_row // NB) * NB

    lane = lax.iota(jnp.int32, 16)
    ones16 = jnp.ones((16,), jnp.int32)

    def row_dma(r, slot, sem):
        return pltpu.make_async_copy(
            d2_hbm.at[pl.ds(base_row + r, 1)], dbuf.at[slot], sem)

    def process(r, slot):
        dref = dbuf.at[slot]

        def comp_body(v, cnt):
            d = dref[0, pl.ds(v * 16, 16)]
            msk = d <= R2
            jvec = lane + v * 16
            plsc.store_compressed(cand_idx.at[pl.ds(cnt, 16)], jvec,
                                  mask=msk)
            plsc.store_compressed(cand_d2.at[pl.ds(cnt, 16)], d, mask=msk)
            return cnt + plsc.all_reduce_population_count(msk)[0]

        m = lax.fori_loop(0, NB // 16, comp_body, jnp.int32(0))

        inf16 = jnp.full((16,), jnp.inf, jnp.float32)
        plsc.store_scatter(cand_d2, [m + lane], inf16)
        nv = (m + 15) // 16

        def do_select():
            pref = jnp.int32(0)
            target = jnp.int32(KNB)
            for pi, shift in enumerate(range(28, -1, -4)):
                hist[pl.ds(0, 16)] = jnp.zeros((16,), jnp.int32)

                def hbody(v, _, shift=shift, pi=pi, pref=pref):
                    d = cand_d2[pl.ds(v * 16, 16)]
                    bits = plsc.bitcast(d, jnp.int32)
                    binv = jnp.right_shift(bits, shift) & 0xF
                    if pi == 0:
                        plsc.addupdate_scatter(hist, [binv], ones16)
                    else:
                        ok = jnp.right_shift(bits, shift + 4) == pref
                        plsc.addupdate_scatter(hist, [binv], ones16,
                                               mask=ok)
                    return 0
                lax.fori_loop(0, nv, hbody, 0)

                hv = hist[pl.ds(0, 16)]
                c = plsc.cumsum(hv)
                ge = c >= target
                np_ = plsc.all_reduce_population_count(ge)[0]
                ffs = 16 - np_
                ebuf[pl.ds(0, 16)] = c
                idxv = lane * 0 + ffs
                c_at = plsc.load_gather(ebuf, [idxv])[0]
                h_at = plsc.load_gather(hist, [idxv])[0]
                cb = c_at - h_at
                if pi == 0:
                    pref = ffs
                else:
                    pref = (pref << 4) | ffs
                target = target - cb
            return pref, target

        t_bits, q = lax.cond(
            m > KNB, do_select,
            lambda: (jnp.int32(INF_BITS), jnp.int32(0)))

        obase = r * KNB
        for v in range(KNB // 16):
            outflat[pl.ds(obase + v * 16, 16)] = jnp.full((16,), SENT,
                                                          jnp.int32)

        def ebody(v, carry):
            ltc, eqc = carry
            d = cand_d2[pl.ds(v * 16, 16)]
            bits = plsc.bitcast(d, jnp.int32)
            lt = bits < t_bits
            eq = bits == t_bits
            gidx = cand_idx[pl.ds(v * 16, 16)] + batch_base
            plsc.store_compressed(outflat.at[pl.ds(obase + ltc, 16)], gidx,
                                  mask=lt)
            plsc.store_compressed(eqbuf.at[pl.ds(eqc, 16)], gidx, mask=eq)
            return (ltc + plsc.all_reduce_population_count(lt)[0],
                    eqc + plsc.all_reduce_population_count(eq)[0])

        ltc, _ = lax.fori_loop(0, nv, ebody, (jnp.int32(0), jnp.int32(0)))

        for v in range(KNB // 16):
            w = eqbuf[pl.ds(v * 16, 16)]
            posv = lane + (v * 16 + ltc + obase)
            keep = (lane + v * 16) < q
            plsc.store_scatter(outflat, [posv], w, mask=keep)

    row_dma(0, 0, sem0).start()
    row_dma(1, 1, sem1).start()

    def pair_body(rr, _):
        r0 = rr * 2
        row_dma(r0, 0, sem0).wait()
        process(r0, 0)

        @pl.when(rr < RW // 2 - 1)
        def _():
            row_dma(r0 + 2, 0, sem0).start()

        row_dma(r0 + 1, 1, sem1).wait()
        process(r0 + 1, 1)

        @pl.when(rr < RW // 2 - 1)
        def _():
            row_dma(r0 + 3, 1, sem1).start()
        return 0

    lax.fori_loop(0, RW // 2, pair_body, 0)

    cp = pltpu.make_async_copy(
        outflat.at[pl.ds(0, RW * KNB)],
        nbr_hbm.at[pl.ds(base_row * KNB, RW * KNB)], osem)
    cp.start()
    cp.wait()


def _nbr_call(d2_all):
    mesh = plsc.VectorSubcoreMesh(core_axis_name="c", subcore_axis_name="s")
    f = pl.kernel(
        _nbr_body,
        mesh=mesh,
        out_type=jax.ShapeDtypeStruct((N * KNB,), jnp.int32),
        scratch_types=[
            pltpu.VMEM((2, 1, NB), jnp.float32),
            pltpu.VMEM((NB + 32,), jnp.int32),
            pltpu.VMEM((NB + 32,), jnp.float32),
            pltpu.VMEM((NB + 32,), jnp.int32),
            pltpu.VMEM((16,), jnp.int32),
            pltpu.VMEM((16,), jnp.int32),
            pltpu.VMEM((RW * KNB + 32,), jnp.int32),
            pltpu.SemaphoreType.DMA,
            pltpu.SemaphoreType.DMA,
            pltpu.SemaphoreType.DMA,
        ],
        compiler_params=pltpu.CompilerParams(needs_layout_passes=False),
    )
    return f(d2_all)


def _gmax_body(a_hbm, c_hbm, nbr_hbm, agg_hbm, nbrbuf, gbuf, cbuf, obuf,
               nsem, csem, sem0, sem1, osem, *, dp):
    wid = lax.axis_index("s") * 2 + lax.axis_index("c")
    base_row = wid * RW

    cpn = pltpu.make_async_copy(
        nbr_hbm.at[pl.ds(base_row * KNB, RW * KNB)], nbrbuf, nsem)
    cpn.start()
    cpc = pltpu.make_async_copy(c_hbm.at[pl.ds(base_row, RW)], cbuf, csem)
    cpc.start()
    cpn.wait()
    cpc.wait()

    def gather_dma(r, slot, sem):
        return pltpu.make_async_copy(
            a_hbm.at[nbrbuf.at[pl.ds((r - base_row) * KNB, KNB)]],
            gbuf.at[slot], sem)

    def process(r, slot):
        rl = r - base_row
        nk = dp // 16
        cvecs = [cbuf[rl, pl.ds(k * 16, 16)] for k in range(nk)]

        def jbody(j8, acc):
            for u in range(8):
                j = j8 * 8 + u
                acc = tuple(
                    jnp.maximum(acc[k],
                                gbuf[slot, j, pl.ds(k * 16, 16)] + cvecs[k])
                    for k in range(nk))
            return acc

        acc0 = tuple(jnp.zeros((16,), jnp.float32) for _ in range(nk))
        acc = lax.fori_loop(0, KNB // 8, jbody, acc0)
        for k in range(nk):
            obuf[rl, pl.ds(k * 16, 16)] = acc[k]

    gather_dma(base_row, 0, sem0).start()
    gather_dma(base_row + 1, 1, sem1).start()

    def pair_body(rr, _):
        r0 = base_row + rr * 2
        gather_dma(r0, 0, sem0).wait()
        process(r0, 0)

        @pl.when(rr < RW // 2 - 1)
        def _():
            gather_dma(r0 + 2, 0, sem0).start()

        gather_dma(r0 + 1, 1, sem1).wait()
        process(r0 + 1, 1)

        @pl.when(rr < RW // 2 - 1)
        def _():
            gather_dma(r0 + 3, 1, sem1).start()
        return 0

    lax.fori_loop(0, RW // 2, pair_body, 0)

    cp = pltpu.make_async_copy(obuf, agg_hbm.at[pl.ds(base_row, RW)], osem)
    cp.start()
    cp.wait()


def _gmax_call(a_table, c_table, nbr, dp):
    mesh = plsc.VectorSubcoreMesh(core_axis_name="c", subcore_axis_name="s")
    f = pl.kernel(
        functools.partial(_gmax_body, dp=dp),
        mesh=mesh,
        out_type=jax.ShapeDtypeStruct((N, dp), jnp.float32),
        scratch_types=[
            pltpu.VMEM((RW * KNB,), jnp.int32),
            pltpu.VMEM((2, KNB, dp), jnp.float32),
            pltpu.VMEM((RW, dp), jnp.float32),
            pltpu.VMEM((RW, dp), jnp.float32),
            pltpu.SemaphoreType.DMA,
            pltpu.SemaphoreType.DMA,
            pltpu.SemaphoreType.DMA,
            pltpu.SemaphoreType.DMA,
            pltpu.SemaphoreType.DMA,
        ],
        compiler_params=pltpu.CompilerParams(
            needs_layout_passes=False, use_tc_tiling_on_sc=False),
    )
    return f(a_table, c_table, nbr)


def _celu(x):
    return jnp.where(x > 0.0, x, jnp.exp(jnp.minimum(x, 0.0)) - 1.0)


def _pad_cols(x, dp):
    c = x.shape[1]
    if c == dp:
        return x
    return jnp.concatenate(
        [x, jnp.zeros((x.shape[0], dp - c), x.dtype)], axis=1)


def _pre_body(pos_ref, w1s_ref, b1_ref, a1_ref, c1_ref):
    p = pos_ref[...]
    w1s = w1s_ref[...]
    b1 = b1_ref[...]
    dp1 = a1_ref.shape[1]
    both = lax.dot_general(p, w1s, (((1,), (0,)), ((), ())),
                           preferred_element_type=jnp.float32)
    a1_ref[pl.ds(0, N), :] = both[:, :dp1] + b1
    a1_ref[pl.ds(N, APAD - N), :] = jnp.full((APAD - N, dp1), -1e30,
                                             jnp.float32)
    c1_ref[...] = both[:, dp1:]


def _pre_call(pos, w1s, b1, dp1):
    return pl.pallas_call(
        _pre_body,
        out_shape=(jax.ShapeDtypeStruct((APAD, dp1), jnp.float32),
                   jax.ShapeDtypeStruct((N, dp1), jnp.float32)),
    )(pos, w1s, b1)


def _step_body(agg_ref, w2_ref, b2_ref, w1n_ref, pn_ref, an_ref, *, cmid):
    agg = agg_ref[...][:, :cmid]
    o = _celu(lax.dot_general(agg, w2_ref[...], (((1,), (0,)), ((), ())),
                              preferred_element_type=jnp.float32)
              + b2_ref[...])
    a_n = lax.dot_general(o, w1n_ref[...], (((1,), (0,)), ((), ())),
                          preferred_element_type=jnp.float32) + pn_ref[...]
    dpn = an_ref.shape[1]
    an_ref[pl.ds(0, N), :] = a_n
    an_ref[pl.ds(N, APAD - N), :] = jnp.full((APAD - N, dpn), -1e30,
                                             jnp.float32)


def _step_call(agg, w2, b2, w1n, pn, cmid, dpn):
    return pl.pallas_call(
        functools.partial(_step_body, cmid=cmid),
        out_shape=jax.ShapeDtypeStruct((APAD, dpn), jnp.float32),
    )(agg, w2, b2, w1n, pn)


def _final_body(agg_ref, w2_ref, b2_ref, out_ref, *, cmid):
    agg = agg_ref[...][:, :cmid]
    out_ref[...] = _celu(
        lax.dot_general(agg, w2_ref[...], (((1,), (0,)), ((), ())),
                        preferred_element_type=jnp.float32) + b2_ref[...])


def _final_call(agg, w2, b2, cmid, cout):
    return pl.pallas_call(
        functools.partial(_final_body, cmid=cmid),
        out_shape=jax.ShapeDtypeStruct((N, cout), jnp.float32),
    )(agg, w2, b2)


def _pn_body(pos_ref, w1p_ref, b1_ref, pn_ref, cn_ref):
    p = pos_ref[...]
    w1p = w1p_ref[...]
    pn = lax.dot_general(p, w1p, (((1,), (0,)), ((), ())),
                         preferred_element_type=jnp.float32)
    pn_ref[...] = pn + b1_ref[...]
    cn_ref[...] = -pn


def _pn_call(pos, w1p, b1, dp):
    return pl.pallas_call(
        _pn_body,
        out_shape=(jax.ShapeDtypeStruct((N, dp), jnp.float32),
                   jax.ShapeDtypeStruct((N, dp), jnp.float32)),
    )(pos, w1p, b1)


def kernel(pos, rgb, batch,
           W1_1, b1_1, W2_1, b2_1,
           W1_2, b1_2, W2_2, b2_2,
           W1_3, b1_3, W2_3, b2_3):
    del rgb
    dp1, dp2, dp3 = 16, 16, 32
    c1, c2, c3 = 8, 16, 32

    d2_all = _d2_call(pos)
    nbr = _nbr_call(d2_all)

    w1a, w1b = W1_1[:3], W1_1[3:]
    w1s = jnp.concatenate(
        [_pad_cols(w1a + w1b, dp1), _pad_cols(-w1b, dp1)], axis=1)
    a1, cc1 = _pre_call(pos, w1s, _pad_cols(b1_1[None, :], dp1), dp1)

    p2, cc2 = _pn_call(pos, _pad_cols(W1_2[c1:], dp2),
                       _pad_cols(b1_2[None, :], dp2), dp2)
    p3, cc3 = _pn_call(pos, _pad_cols(W1_3[c2:], dp3),
                       _pad_cols(b1_3[None, :], dp3), dp3)

    agg1 = _gmax_call(a1, cc1, nbr, dp1)
    a2 = _step_call(agg1, W2_1, b2_1[None, :],
                    _pad_cols(W1_2[:c1], dp2), p2, c1, dp2)
    agg2 = _gmax_call(a2, cc2, nbr, dp2)
    a3 = _step_call(agg2, W2_2, b2_2[None, :],
                    _pad_cols(W1_3[:c2], dp3), p3, c2, dp3)
    agg3 = _gmax_call(a3, cc3, nbr, dp3)
    out = _final_call(agg3, W2_3, b2_3[None, :], c3, c3)

    return (pos, out, batch)


# speedup vs baseline: 59.6518x; 1.4344x over previous
import functools

import jax
import jax.numpy as jnp
from jax import lax
from jax.experimental import pallas as pl
from jax.experimental.pallas import tpu as pltpu
from jax.experimental.pallas import tpu_sc as plsc

N = 16384
B = 8
NB = N // B
KNB = 64
R2 = (1.0 / 16.0) ** 2
NWORK = 32
RW = N // NWORK
GR = 8
SENT = N
APAD = N + 128
INF_BITS = 0x7F800000


def _d2_body(pos_ref, o_ref):
    p = pos_ref[...]
    x, y, z = p[:, 0], p[:, 1], p[:, 2]
    sq = (x * x + z * z) + y * y
    pb = p.astype(jnp.bfloat16)
    dot = lax.dot_general(pb, pb, (((1,), (1,)), ((), ())),
                          preferred_element_type=jnp.float32)
    o_ref[...] = jnp.maximum(sq[:, None] + sq[None, :] - 2.0 * dot, 0.0)


def _d2_call(pos):
    return pl.pallas_call(
        _d2_body,
        grid=(B,),
        in_specs=[pl.BlockSpec((NB, 3), lambda b: (b, 0))],
        out_specs=pl.BlockSpec((NB, NB), lambda b: (b, 0)),
        out_shape=jax.ShapeDtypeStruct((N, NB), jnp.float32),
    )(pos)


def _nbr_body(d2_hbm, nbr_hbm, dbuf, cand_idx, cand_d2, eqbuf, hist, ebuf,
              outflat, sem0, sem1, osem):
    wid = lax.axis_index("s") * 2 + lax.axis_index("c")
    base_row = wid * RW
    batch_base = (base_row // NB) * NB

    lane = lax.iota(jnp.int32, 16)
    ones16 = jnp.ones((16,), jnp.int32)

    def row_dma(g, slot, sem):
        return pltpu.make_async_copy(
            d2_hbm.at[pl.ds(base_row + g * GR, GR)], dbuf.at[slot], sem)

    def process(r, slot, u):
        dref = dbuf.at[slot]

        def comp_body(v, cnt):
            d = dref[u, pl.ds(v * 16, 16)]
            msk = d <= R2
            jvec = lane + v * 16
            plsc.store_compressed(cand_idx.at[pl.ds(cnt, 16)], jvec,
                                  mask=msk)
            plsc.store_compressed(cand_d2.at[pl.ds(cnt, 16)], d, mask=msk)
            return cnt + plsc.all_reduce_population_count(msk)[0]

        m = lax.fori_loop(0, NB // 16, comp_body, jnp.int32(0))

        inf16 = jnp.full((16,), jnp.inf, jnp.float32)
        plsc.store_scatter(cand_d2, [m + lane], inf16)
        nv = (m + 15) // 16

        def do_select():
            pref = jnp.int32(0)
            target = jnp.int32(KNB)
            for pi, shift in enumerate(range(28, -1, -4)):
                hist[pl.ds(0, 16)] = jnp.zeros((16,), jnp.int32)

                def hbody(v, _, shift=shift, pi=pi, pref=pref):
                    d = cand_d2[pl.ds(v * 16, 16)]
                    bits = plsc.bitcast(d, jnp.int32)
                    binv = jnp.right_shift(bits, shift) & 0xF
                    if pi == 0:
                        plsc.addupdate_scatter(hist, [binv], ones16)
                    else:
                        ok = jnp.right_shift(bits, shift + 4) == pref
                        plsc.addupdate_scatter(hist, [binv], ones16,
                                               mask=ok)
                    return 0
                lax.fori_loop(0, nv, hbody, 0)

                hv = hist[pl.ds(0, 16)]
                c = plsc.cumsum(hv)
                ge = c >= target
                np_ = plsc.all_reduce_population_count(ge)[0]
                ffs = 16 - np_
                ebuf[pl.ds(0, 16)] = c
                idxv = lane * 0 + ffs
                c_at = plsc.load_gather(ebuf, [idxv])[0]
                h_at = plsc.load_gather(hist, [idxv])[0]
                cb = c_at - h_at
                if pi == 0:
                    pref = ffs
                else:
                    pref = (pref << 4) | ffs
                target = target - cb
            return pref, target

        t_bits, q = lax.cond(
            m > KNB, do_select,
            lambda: (jnp.int32(INF_BITS), jnp.int32(0)))

        obase = r * KNB
        for v in range(KNB // 16):
            outflat[pl.ds(obase + v * 16, 16)] = jnp.full((16,), SENT,
                                                          jnp.int32)

        def ebody(v, carry):
            ltc, eqc = carry
            d = cand_d2[pl.ds(v * 16, 16)]
            bits = plsc.bitcast(d, jnp.int32)
            lt = bits < t_bits
            eq = bits == t_bits
            gidx = cand_idx[pl.ds(v * 16, 16)] + batch_base
            plsc.store_compressed(outflat.at[pl.ds(obase + ltc, 16)], gidx,
                                  mask=lt)
            plsc.store_compressed(eqbuf.at[pl.ds(eqc, 16)], gidx, mask=eq)
            return (ltc + plsc.all_reduce_population_count(lt)[0],
                    eqc + plsc.all_reduce_population_count(eq)[0])

        ltc, _ = lax.fori_loop(0, nv, ebody, (jnp.int32(0), jnp.int32(0)))

        for v in range(KNB // 16):
            w = eqbuf[pl.ds(v * 16, 16)]
            posv = lane + (v * 16 + ltc + obase)
            keep = (lane + v * 16) < q
            plsc.store_scatter(outflat, [posv], w, mask=keep)

    NG = RW // GR
    row_dma(0, 0, sem0).start()
    row_dma(1, 1, sem1).start()

    def pair_body(gg, _):
        g0 = gg * 2

        def rows0(uu, _):
            process(g0 * GR + uu, 0, uu)
            return 0

        def rows1(uu, _):
            process((g0 + 1) * GR + uu, 1, uu)
            return 0

        row_dma(g0, 0, sem0).wait()
        lax.fori_loop(0, GR, rows0, 0)

        @pl.when(gg < NG // 2 - 1)
        def _():
            row_dma(g0 + 2, 0, sem0).start()

        row_dma(g0 + 1, 1, sem1).wait()
        lax.fori_loop(0, GR, rows1, 0)

        @pl.when(gg < NG // 2 - 1)
        def _():
            row_dma(g0 + 3, 1, sem1).start()
        return 0

    lax.fori_loop(0, NG // 2, pair_body, 0)

    cp = pltpu.make_async_copy(
        outflat.at[pl.ds(0, RW * KNB)],
        nbr_hbm.at[pl.ds(base_row * KNB, RW * KNB)], osem)
    cp.start()
    cp.wait()


def _nbr_call(d2_all):
    mesh = plsc.VectorSubcoreMesh(core_axis_name="c", subcore_axis_name="s")
    f = pl.kernel(
        _nbr_body,
        mesh=mesh,
        out_type=jax.ShapeDtypeStruct((N * KNB,), jnp.int32),
        scratch_types=[
            pltpu.VMEM((2, GR, NB), jnp.float32),
            pltpu.VMEM((NB + 32,), jnp.int32),
            pltpu.VMEM((NB + 32,), jnp.float32),
            pltpu.VMEM((NB + 32,), jnp.int32),
            pltpu.VMEM((16,), jnp.int32),
            pltpu.VMEM((16,), jnp.int32),
            pltpu.VMEM((RW * KNB + 32,), jnp.int32),
            pltpu.SemaphoreType.DMA,
            pltpu.SemaphoreType.DMA,
            pltpu.SemaphoreType.DMA,
        ],
        compiler_params=pltpu.CompilerParams(needs_layout_passes=False),
    )
    return f(d2_all)


def _gmax_body(a_hbm, c_hbm, nbr_hbm, agg_hbm, abuf, nbrbuf, cbuf, obuf,
               asem, nsem, csem, osem, *, dp):
    wid = lax.axis_index("s") * 2 + lax.axis_index("c")
    base_row = wid * RW
    batch_base = (base_row // NB) * NB
    nk = dp // 16
    RH = RW // 2

    cpa = pltpu.make_async_copy(
        a_hbm.at[pl.ds(batch_base * dp, NB * dp)],
        abuf.at[pl.ds(0, NB * dp)], asem)
    cpa.start()
    cpc = pltpu.make_async_copy(
        c_hbm.at[pl.ds(base_row * dp, RW * dp)], cbuf, csem)
    cpc.start()

    def nbr_dma(half):
        return pltpu.make_async_copy(
            nbr_hbm.at[pl.ds((base_row + half * RH) * KNB, RH * KNB)],
            nbrbuf, nsem)

    nbr_dma(0).start()
    cpa.wait()
    for k in range(nk):
        abuf[pl.ds(NB * dp + k * 16, 16)] = jnp.full((16,), -1e30,
                                                     jnp.float32)
    cpc.wait()

    def do_row(rl, rlc):
        cvecs = [cbuf[pl.ds(rl * dp + k * 16, 16)] for k in range(nk)]
        acc = [jnp.zeros((16,), jnp.float32) for _ in range(nk)]
        for v in range(KNB // 16):
            w = nbrbuf[pl.ds(rlc * KNB + v * 16, 16)]
            off = jnp.minimum(w - batch_base, NB) * dp
            for u in range(16):
                o = off[u]
                for k in range(nk):
                    acc[k] = jnp.maximum(acc[k],
                                         abuf[pl.ds(o + k * 16, 16)]
                                         + cvecs[k])
        for k in range(nk):
            obuf[pl.ds(rl * dp + k * 16, 16)] = acc[k]

    for half in range(2):
        nbr_dma(half).wait()

        def rbody(rlc, _, half=half):
            do_row(half * RH + rlc, rlc)
            return 0

        lax.fori_loop(0, RH, rbody, 0)
        if half == 0:
            nbr_dma(1).start()

    cp = pltpu.make_async_copy(
        obuf, agg_hbm.at[pl.ds(base_row * dp, RW * dp)], osem)
    cp.start()
    cp.wait()


def _gmax_call(a_table, c_table, nbr, dp):
    mesh = plsc.VectorSubcoreMesh(core_axis_name="c", subcore_axis_name="s")
    f = pl.kernel(
        functools.partial(_gmax_body, dp=dp),
        mesh=mesh,
        out_type=jax.ShapeDtypeStruct((N * dp,), jnp.float32),
        scratch_types=[
            pltpu.VMEM(((NB + 8) * dp,), jnp.float32),
            pltpu.VMEM((RW // 2 * KNB,), jnp.int32),
            pltpu.VMEM((RW * dp,), jnp.float32),
            pltpu.VMEM((RW * dp,), jnp.float32),
            pltpu.SemaphoreType.DMA,
            pltpu.SemaphoreType.DMA,
            pltpu.SemaphoreType.DMA,
            pltpu.SemaphoreType.DMA,
        ],
        compiler_params=pltpu.CompilerParams(
            needs_layout_passes=False, use_tc_tiling_on_sc=False),
    )
    return f(a_table.reshape(-1), c_table.reshape(-1), nbr)


def _celu(x):
    return jnp.where(x > 0.0, x, jnp.exp(jnp.minimum(x, 0.0)) - 1.0)


def _pad_cols(x, dp):
    c = x.shape[1]
    if c == dp:
        return x
    return jnp.concatenate(
        [x, jnp.zeros((x.shape[0], dp - c), x.dtype)], axis=1)


def _pre_body(pos_ref, w1s_ref, b1_ref, a1_ref, c1_ref):
    p = pos_ref[...]
    w1s = w1s_ref[...]
    b1 = b1_ref[...]
    dp1 = a1_ref.shape[1]
    both = lax.dot_general(p, w1s, (((1,), (0,)), ((), ())),
                           preferred_element_type=jnp.float32)
    a1_ref[pl.ds(0, N), :] = both[:, :dp1] + b1
    a1_ref[pl.ds(N, APAD - N), :] = jnp.full((APAD - N, dp1), -1e30,
                                             jnp.float32)
    c1_ref[...] = both[:, dp1:]


def _pre_call(pos, w1s, b1, dp1):
    return pl.pallas_call(
        _pre_body,
        out_shape=(jax.ShapeDtypeStruct((APAD, dp1), jnp.float32),
                   jax.ShapeDtypeStruct((N, dp1), jnp.float32)),
    )(pos, w1s, b1)


def _step_body(agg_ref, w2_ref, b2_ref, w1n_ref, pn_ref, an_ref, *, cmid):
    agg = agg_ref[...][:, :cmid]
    o = _celu(lax.dot_general(agg, w2_ref[...], (((1,), (0,)), ((), ())),
                              preferred_element_type=jnp.float32)
              + b2_ref[...])
    a_n = lax.dot_general(o, w1n_ref[...], (((1,), (0,)), ((), ())),
                          preferred_element_type=jnp.float32) + pn_ref[...]
    dpn = an_ref.shape[1]
    an_ref[pl.ds(0, N), :] = a_n
    an_ref[pl.ds(N, APAD - N), :] = jnp.full((APAD - N, dpn), -1e30,
                                             jnp.float32)


def _step_call(agg, w2, b2, w1n, pn, cmid, dpn):
    return pl.pallas_call(
        functools.partial(_step_body, cmid=cmid),
        out_shape=jax.ShapeDtypeStruct((APAD, dpn), jnp.float32),
    )(agg, w2, b2, w1n, pn)


def _final_body(agg_ref, w2_ref, b2_ref, out_ref, *, cmid):
    agg = agg_ref[...][:, :cmid]
    out_ref[...] = _celu(
        lax.dot_general(agg, w2_ref[...], (((1,), (0,)), ((), ())),
                        preferred_element_type=jnp.float32) + b2_ref[...])


def _final_call(agg, w2, b2, cmid, cout):
    return pl.pallas_call(
        functools.partial(_final_body, cmid=cmid),
        out_shape=jax.ShapeDtypeStruct((N, cout), jnp.float32),
    )(agg, w2, b2)


def _pn_body(pos_ref, w1p_ref, b1_ref, pn_ref, cn_ref):
    p = pos_ref[...]
    w1p = w1p_ref[...]
    pn = lax.dot_general(p, w1p, (((1,), (0,)), ((), ())),
                         preferred_element_type=jnp.float32)
    pn_ref[...] = pn + b1_ref[...]
    cn_ref[...] = -pn


def _pn_call(pos, w1p, b1, dp):
    return pl.pallas_call(
        _pn_body,
        out_shape=(jax.ShapeDtypeStruct((N, dp), jnp.float32),
                   jax.ShapeDtypeStruct((N, dp), jnp.float32)),
    )(pos, w1p, b1)


def kernel(pos, rgb, batch,
           W1_1, b1_1, W2_1, b2_1,
           W1_2, b1_2, W2_2, b2_2,
           W1_3, b1_3, W2_3, b2_3):
    del rgb
    dp1, dp2, dp3 = 16, 16, 32
    c1, c2, c3 = 8, 16, 32

    d2_all = _d2_call(pos)
    nbr = _nbr_call(d2_all)

    w1a, w1b = W1_1[:3], W1_1[3:]
    w1s = jnp.concatenate(
        [_pad_cols(w1a + w1b, dp1), _pad_cols(-w1b, dp1)], axis=1)
    a1, cc1 = _pre_call(pos, w1s, _pad_cols(b1_1[None, :], dp1), dp1)

    p2, cc2 = _pn_call(pos, _pad_cols(W1_2[c1:], dp2),
                       _pad_cols(b1_2[None, :], dp2), dp2)
    p3, cc3 = _pn_call(pos, _pad_cols(W1_3[c2:], dp3),
                       _pad_cols(b1_3[None, :], dp3), dp3)

    agg1 = _gmax_call(a1, cc1, nbr, dp1).reshape(N, dp1)
    a2 = _step_call(agg1, W2_1, b2_1[None, :],
                    _pad_cols(W1_2[:c1], dp2), p2, c1, dp2)
    agg2 = _gmax_call(a2, cc2, nbr, dp2).reshape(N, dp2)
    a3 = _step_call(agg2, W2_2, b2_2[None, :],
                    _pad_cols(W1_3[:c2], dp3), p3, c2, dp3)
    agg3 = _gmax_call(a3, cc3, nbr, dp3).reshape(N, dp3)
    out = _final_call(agg3, W2_3, b2_3[None, :], c3, c3)

    return (pos, out, batch)
